# baseline (speedup 1.0000x reference)
import functools

import jax
import jax.numpy as jnp
from jax import lax
from jax.experimental import pallas as pl
from jax.experimental.pallas import tpu as pltpu
from jax.experimental.pallas import tpu_sc as plsc

N = 10000
E = 320000
D = 128
NG = 64
NC, NS = 2, 16
NW = NC * NS
K = 128
NP = 10240
EP = 327680
CPT = EP // K // NW
ARPT = NP // NS
PRPT = 320

_mesh = plsc.VectorSubcoreMesh(
    core_axis_name="c", subcore_axis_name="s", num_cores=NC, num_subcores=NS)


def _zero_1d(ref, nwords, unroll=4):
    z = jnp.zeros((16,), jnp.float32)

    def body(i, _):
        ref[pl.ds(i * 16, 16)] = z
        return 0

    lax.fori_loop(0, nwords // 16, body, 0, unroll=unroll)


@functools.partial(
    pl.kernel,
    out_type=jax.ShapeDtypeStruct((NC, NP, D), jnp.float32),
    mesh=_mesh,
    scratch_types=[
        pltpu.VMEM((CPT // 2, K), jnp.int32),
        pltpu.VMEM((CPT // 2, K), jnp.int32),
        pltpu.VMEM((K, D), jnp.float32),
        pltpu.VMEM((K, D), jnp.float32),
        pltpu.VMEM_SHARED((NP, D), jnp.float32),
        pltpu.SemaphoreType.DMA,
        pltpu.SemaphoreType.DMA,
        pltpu.SemaphoreType.DMA,
        pltpu.SemaphoreType.DMA,
    ],
)
def _agg_sc(h_hbm, src_hbm, dst_hbm, out_hbm,
            src_v, dst_v, rows_a, rows_b, acc, sem_a, sem_b, sem_sa, sem_sb):
    c = lax.axis_index("c")
    s = lax.axis_index("s")
    w = s * NC + c

    def zrow(i, _):
        row = rows_a.at[i]
        for j in range(D // 16):
            row[pl.ds(j * 16, 16)] = jnp.zeros((16,), jnp.float32)
        return 0

    lax.fori_loop(0, K, zrow, 0)

    def zacc(j, _):
        pltpu.sync_copy(rows_a, acc.at[pl.ds(s * ARPT + j * K, K)])
        return 0

    lax.fori_loop(0, ARPT // K, zacc, 0)

    plsc.subcore_barrier()

    base = w * CPT
    HCPT = CPT // 2
    for half in range(2):
        hb = base + half * HCPT
        pltpu.sync_copy(src_hbm.at[pl.ds(hb, HCPT)], src_v)
        pltpu.sync_copy(dst_hbm.at[pl.ds(hb, HCPT)], dst_v)
        pltpu.async_copy(h_hbm.at[src_v.at[0]], rows_a, sem_a)
        pltpu.async_copy(h_hbm.at[src_v.at[1]], rows_b, sem_b)

        def chunk(jj, _):
            j0 = 2 * jj
            pltpu.make_async_copy(
                h_hbm.at[src_v.at[j0]], rows_a, sem_a).wait()
            pltpu.async_copy(rows_a, acc.at[dst_v.at[j0]], sem_sa, add=True)
            pltpu.make_async_copy(
                h_hbm.at[src_v.at[j0 + 1]], rows_b, sem_b).wait()
            pltpu.async_copy(rows_b, acc.at[dst_v.at[j0 + 1]], sem_sb,
                             add=True)

            @pl.when(jj < HCPT // 2 - 1)
            def _():
                pltpu.make_async_copy(
                    rows_a, acc.at[dst_v.at[j0]], sem_sa).wait()
                pltpu.async_copy(h_hbm.at[src_v.at[j0 + 2]], rows_a, sem_a)
                pltpu.make_async_copy(
                    rows_b, acc.at[dst_v.at[j0 + 1]], sem_sb).wait()
                pltpu.async_copy(h_hbm.at[src_v.at[j0 + 3]], rows_b, sem_b)

            return 0

        lax.fori_loop(0, HCPT // 2, chunk, 0)
        pltpu.make_async_copy(
            rows_a, acc.at[dst_v.at[HCPT - 2]], sem_sa).wait()
        pltpu.make_async_copy(
            rows_b, acc.at[dst_v.at[HCPT - 1]], sem_sb).wait()

    plsc.subcore_barrier()

    def flush(j, _):
        r0 = s * ARPT + j * K
        pltpu.sync_copy(acc.at[pl.ds(r0, K)], rows_a)
        pltpu.sync_copy(rows_a, out_hbm.at[c].at[pl.ds(r0, K)])
        return 0

    lax.fori_loop(0, ARPT // K, flush, 0)


def _layer_tc(h, parts, W, b):
    BM = 2000

    def body(h_ref, p_ref, w_ref, b_ref, o_ref):
        acc = h_ref[...] + p_ref[0] + p_ref[1]
        y = jnp.dot(acc, w_ref[...], preferred_element_type=jnp.float32)
        o_ref[...] = jnp.maximum(y + b_ref[...], 0.0)

    return pl.pallas_call(
        body,
        grid=(N // BM,),
        in_specs=[
            pl.BlockSpec((BM, D), lambda i: (i, 0)),
            pl.BlockSpec((NC, BM, D), lambda i: (0, i, 0)),
            pl.BlockSpec((D, D), lambda i: (0, 0)),
            pl.BlockSpec((1, D), lambda i: (0, 0)),
        ],
        out_specs=pl.BlockSpec((BM, D), lambda i: (i, 0)),
        out_shape=jax.ShapeDtypeStruct((N, D), jnp.float32),
    )(h, parts, W, b.reshape(1, D))


@functools.partial(
    pl.kernel,
    out_type=[
        jax.ShapeDtypeStruct((NW, NG * D), jnp.float32),
        jax.ShapeDtypeStruct((NW, NG * D), jnp.float32),
        jax.ShapeDtypeStruct((NW, 128), jnp.float32),
    ],
    mesh=_mesh,
    scratch_types=[
        pltpu.VMEM((PRPT, D), jnp.float32),
        pltpu.VMEM((PRPT + 16,), jnp.int32),
        pltpu.VMEM((NG * D,), jnp.float32),
        pltpu.VMEM((NG * D,), jnp.float32),
        pltpu.VMEM((128 + 16,), jnp.float32),
    ],
)
def _pool_sc(h_hbm, ids_hbm, omax, osum, ocnt,
             rows_v, ids_v, lmax, lsum, lcnt):
    c = lax.axis_index("c")
    s = lax.axis_index("s")
    w = s * NC + c
    row0 = w * PRPT
    nlast = N - (NW - 1) * PRPT
    nrows = jnp.where(w == NW - 1, nlast, PRPT)

    neg = jnp.full((16,), -jnp.inf, jnp.float32)

    def iminf(i, _):
        lmax[pl.ds(i * 16, 16)] = neg
        return 0

    lax.fori_loop(0, NG * D // 16, iminf, 0, unroll=4)
    _zero_1d(lsum, NG * D)
    _zero_1d(lcnt, 128 + 16)

    @pl.when(w < NW - 1)
    def _():
        pltpu.sync_copy(h_hbm.at[pl.ds(row0, PRPT)], rows_v)
        pltpu.sync_copy(ids_hbm.at[pl.ds(row0, PRPT)],
                        ids_v.at[pl.ds(0, PRPT)])

    @pl.when(w == NW - 1)
    def _():
        pltpu.sync_copy(h_hbm.at[pl.ds(row0, nlast)],
                        rows_v.at[pl.ds(0, nlast)])
        pltpu.sync_copy(ids_hbm.at[pl.ds(row0, nlast)],
                        ids_v.at[pl.ds(0, nlast)])

    lane0 = jnp.where(
        lax.broadcasted_iota(jnp.int32, (16,), 0) == 0, 1.0, 0.0
    ).astype(jnp.float32)

    def rbody(i, _):
        bid = ids_v[pl.ds(i, 16)][0]
        bse = bid * D
        lcnt[pl.ds(bid, 16)] = lcnt[pl.ds(bid, 16)] + lane0
        for j in range(D // 16):
            v = rows_v[i, pl.ds(j * 16, 16)]
            m = lmax[pl.ds(bse + j * 16, 16)]
            lmax[pl.ds(bse + j * 16, 16)] = jnp.maximum(m, v)
            sm = lsum[pl.ds(bse + j * 16, 16)]
            lsum[pl.ds(bse + j * 16, 16)] = sm + v
        return 0

    lax.fori_loop(0, nrows, rbody, 0)

    pltpu.sync_copy(lmax, omax.at[w])
    pltpu.sync_copy(lsum, osum.at[w])
    pltpu.sync_copy(lcnt.at[pl.ds(0, 128)], ocnt.at[w])


def _head_tc(pmax, psum, pcnt, Wl1, bl1, Wl2, bl2, Wl3, bl3):
    def body(pm, ps, pc, w1, b1, w2, b2, w3, b3, o_ref):
        m = pm[0]
        sm = ps[0]
        cnt = pc[0]
        for i in range(1, NW):
            m = jnp.maximum(m, pm[i])
            sm = sm + ps[i]
            cnt = cnt + pc[i]
        m = jnp.where(m > -1e30, m, 0.0)
        mean = sm / jnp.maximum(cnt, 1.0)
        g = jnp.concatenate([m, mean], axis=1)
        g = jnp.maximum(
            jnp.dot(g, w1[...], preferred_element_type=jnp.float32) + b1[...],
            0.0)
        g = jnp.maximum(
            jnp.dot(g, w2[...], preferred_element_type=jnp.float32) + b2[...],
            0.0)
        o_ref[...] = (
            jnp.dot(g, w3[...], preferred_element_type=jnp.float32) + b3[...])

    return pl.pallas_call(
        body,
        out_shape=jax.ShapeDtypeStruct((NG, 1), jnp.float32),
    )(pmax.reshape(NW, NG, D), psum.reshape(NW, NG, D),
      pcnt[:, :NG].reshape(NW, NG, 1),
      Wl1, bl1.reshape(1, -1), Wl2, bl2.reshape(1, -1),
      Wl3, bl3.reshape(1, -1))


def kernel(x, edge_attr, edge_index, batch_index,
           W1, b1, W2, b2, W3, b3, Wl1, bl1, Wl2, bl2, Wl3, bl3):
    del edge_attr

    pad = EP - E
    ar = jnp.arange(pad, dtype=jnp.int32)
    src_pad = (ar * 997) % N
    dst_pad = N + (ar % (NP - N))
    srcp = jnp.concatenate([edge_index[0], src_pad]).reshape(EP // K, K)
    dstp = jnp.concatenate([edge_index[1], dst_pad]).reshape(EP // K, K)

    h = x
    for (W, b) in ((W1, b1), (W2, b2), (W3, b3)):
        parts = _agg_sc(h, srcp, dstp)
        h = _layer_tc(h, parts, W, b)

    pmax, psum, pcnt = _pool_sc(h, batch_index)
    return _head_tc(pmax, psum, pcnt, Wl1, bl1, Wl2, bl2, Wl3, bl3)


# speedup vs baseline: 1.0964x; 1.0964x over previous
import functools

import jax
import jax.numpy as jnp
from jax import lax
from jax.experimental import pallas as pl
from jax.experimental.pallas import tpu as pltpu
from jax.experimental.pallas import tpu_sc as plsc

N = 10000
E = 320000
D = 128
NG = 64
NC, NS = 2, 16
NW = NC * NS
K = 128
NP = 10240
EP = 327680
CPT = EP // K // NW
ARPT = NP // NS
PRPT = 320

_mesh = plsc.VectorSubcoreMesh(
    core_axis_name="c", subcore_axis_name="s", num_cores=NC, num_subcores=NS)


def _zero_1d(ref, nwords, unroll=4):
    z = jnp.zeros((16,), jnp.float32)

    def body(i, _):
        ref[pl.ds(i * 16, 16)] = z
        return 0

    lax.fori_loop(0, nwords // 16, body, 0, unroll=unroll)


@functools.partial(
    pl.kernel,
    out_type=jax.ShapeDtypeStruct((NC, NP, D), jnp.float32),
    mesh=_mesh,
    scratch_types=[
        pltpu.VMEM((CPT // 2, K), jnp.int32),
        pltpu.VMEM((CPT // 2, K), jnp.int32),
        pltpu.VMEM((K, D), jnp.float32),
        pltpu.VMEM((K, D), jnp.float32),
        pltpu.VMEM_SHARED((NP, D), jnp.float32),
        pltpu.SemaphoreType.DMA,
        pltpu.SemaphoreType.DMA,
    ],
)
def _agg_sc(h_hbm, src_hbm, dst_hbm, out_hbm,
            src_v, dst_v, rows_a, rows_b, acc, sem_a, sem_b):
    c = lax.axis_index("c")
    s = lax.axis_index("s")
    w = s * NC + c

    def zrow(i, _):
        row = rows_a.at[i]
        for j in range(D // 16):
            row[pl.ds(j * 16, 16)] = jnp.zeros((16,), jnp.float32)
        return 0

    lax.fori_loop(0, K, zrow, 0)

    def zacc(j, _):
        pltpu.sync_copy(rows_a, acc.at[pl.ds(s * ARPT + j * K, K)])
        return 0

    lax.fori_loop(0, ARPT // K, zacc, 0)

    plsc.subcore_barrier()

    base = w * CPT
    HCPT = CPT // 2
    for half in range(2):
        hb = base + half * HCPT
        pltpu.sync_copy(src_hbm.at[pl.ds(hb, HCPT)], src_v)
        pltpu.sync_copy(dst_hbm.at[pl.ds(hb, HCPT)], dst_v)
        pltpu.async_copy(h_hbm.at[src_v.at[0]], rows_a, sem_a)

        def chunk(jj, _):
            j0 = 2 * jj
            pltpu.make_async_copy(
                h_hbm.at[src_v.at[j0]], rows_a, sem_a).wait()
            pltpu.async_copy(h_hbm.at[src_v.at[j0 + 1]], rows_b, sem_b)
            pltpu.sync_copy(rows_a, acc.at[dst_v.at[j0]], add=True)
            pltpu.make_async_copy(
                h_hbm.at[src_v.at[j0 + 1]], rows_b, sem_b).wait()

            @pl.when(jj < HCPT // 2 - 1)
            def _():
                pltpu.async_copy(h_hbm.at[src_v.at[j0 + 2]], rows_a, sem_a)

            pltpu.sync_copy(rows_b, acc.at[dst_v.at[j0 + 1]], add=True)
            return 0

        lax.fori_loop(0, HCPT // 2, chunk, 0)

    plsc.subcore_barrier()

    def flush(j, _):
        r0 = s * ARPT + j * K
        pltpu.sync_copy(acc.at[pl.ds(r0, K)], rows_a)
        pltpu.sync_copy(rows_a, out_hbm.at[c].at[pl.ds(r0, K)])
        return 0

    lax.fori_loop(0, ARPT // K, flush, 0)


def _layer_tc(h, parts, W, b):
    BM = 2000

    def body(h_ref, p_ref, w_ref, b_ref, o_ref):
        acc = h_ref[...] + p_ref[0] + p_ref[1]
        y = jnp.dot(acc, w_ref[...], preferred_element_type=jnp.float32)
        o_ref[...] = jnp.maximum(y + b_ref[...], 0.0)

    return pl.pallas_call(
        body,
        grid=(N // BM,),
        in_specs=[
            pl.BlockSpec((BM, D), lambda i: (i, 0)),
            pl.BlockSpec((NC, BM, D), lambda i: (0, i, 0)),
            pl.BlockSpec((D, D), lambda i: (0, 0)),
            pl.BlockSpec((1, D), lambda i: (0, 0)),
        ],
        out_specs=pl.BlockSpec((BM, D), lambda i: (i, 0)),
        out_shape=jax.ShapeDtypeStruct((N, D), jnp.float32),
    )(h, parts, W, b.reshape(1, D))


@functools.partial(
    pl.kernel,
    out_type=[
        jax.ShapeDtypeStruct((NW, NG * D), jnp.float32),
        jax.ShapeDtypeStruct((NW, NG * D), jnp.float32),
        jax.ShapeDtypeStruct((NW, 128), jnp.float32),
    ],
    mesh=_mesh,
    scratch_types=[
        pltpu.VMEM((PRPT, D), jnp.float32),
        pltpu.VMEM((PRPT + 16,), jnp.int32),
        pltpu.VMEM((NG * D,), jnp.float32),
        pltpu.VMEM((NG * D,), jnp.float32),
        pltpu.VMEM((128 + 16,), jnp.float32),
    ],
)
def _pool_sc(h_hbm, ids_hbm, omax, osum, ocnt,
             rows_v, ids_v, lmax, lsum, lcnt):
    c = lax.axis_index("c")
    s = lax.axis_index("s")
    w = s * NC + c
    row0 = w * PRPT
    nlast = N - (NW - 1) * PRPT
    nrows = jnp.where(w == NW - 1, nlast, PRPT)

    neg = jnp.full((16,), -jnp.inf, jnp.float32)

    def iminf(i, _):
        lmax[pl.ds(i * 16, 16)] = neg
        return 0

    lax.fori_loop(0, NG * D // 16, iminf, 0, unroll=4)
    _zero_1d(lsum, NG * D)
    _zero_1d(lcnt, 128 + 16)

    @pl.when(w < NW - 1)
    def _():
        pltpu.sync_copy(h_hbm.at[pl.ds(row0, PRPT)], rows_v)
        pltpu.sync_copy(ids_hbm.at[pl.ds(row0, PRPT)],
                        ids_v.at[pl.ds(0, PRPT)])

    @pl.when(w == NW - 1)
    def _():
        pltpu.sync_copy(h_hbm.at[pl.ds(row0, nlast)],
                        rows_v.at[pl.ds(0, nlast)])
        pltpu.sync_copy(ids_hbm.at[pl.ds(row0, nlast)],
                        ids_v.at[pl.ds(0, nlast)])

    lane0 = jnp.where(
        lax.broadcasted_iota(jnp.int32, (16,), 0) == 0, 1.0, 0.0
    ).astype(jnp.float32)

    def rbody(i, _):
        bid = ids_v[pl.ds(i, 16)][0]
        bse = bid * D
        lcnt[pl.ds(bid, 16)] = lcnt[pl.ds(bid, 16)] + lane0
        for j in range(D // 16):
            v = rows_v[i, pl.ds(j * 16, 16)]
            m = lmax[pl.ds(bse + j * 16, 16)]
            lmax[pl.ds(bse + j * 16, 16)] = jnp.maximum(m, v)
            sm = lsum[pl.ds(bse + j * 16, 16)]
            lsum[pl.ds(bse + j * 16, 16)] = sm + v
        return 0

    lax.fori_loop(0, nrows, rbody, 0)

    pltpu.sync_copy(lmax, omax.at[w])
    pltpu.sync_copy(lsum, osum.at[w])
    pltpu.sync_copy(lcnt.at[pl.ds(0, 128)], ocnt.at[w])


def _head_tc(pmax, psum, pcnt, Wl1, bl1, Wl2, bl2, Wl3, bl3):
    def body(pm, ps, pc, w1, b1, w2, b2, w3, b3, o_ref):
        m = pm[0]
        sm = ps[0]
        cnt = pc[0]
        for i in range(1, NW):
            m = jnp.maximum(m, pm[i])
            sm = sm + ps[i]
            cnt = cnt + pc[i]
        m = jnp.where(m > -1e30, m, 0.0)
        mean = sm / jnp.maximum(cnt, 1.0)
        g = jnp.concatenate([m, mean], axis=1)
        g = jnp.maximum(
            jnp.dot(g, w1[...], preferred_element_type=jnp.float32) + b1[...],
            0.0)
        g = jnp.maximum(
            jnp.dot(g, w2[...], preferred_element_type=jnp.float32) + b2[...],
            0.0)
        o_ref[...] = (
            jnp.dot(g, w3[...], preferred_element_type=jnp.float32) + b3[...])

    return pl.pallas_call(
        body,
        out_shape=jax.ShapeDtypeStruct((NG, 1), jnp.float32),
    )(pmax.reshape(NW, NG, D), psum.reshape(NW, NG, D),
      pcnt[:, :NG].reshape(NW, NG, 1),
      Wl1, bl1.reshape(1, -1), Wl2, bl2.reshape(1, -1),
      Wl3, bl3.reshape(1, -1))


def kernel(x, edge_attr, edge_index, batch_index,
           W1, b1, W2, b2, W3, b3, Wl1, bl1, Wl2, bl2, Wl3, bl3):
    del edge_attr

    pad = EP - E
    ar = jnp.arange(pad, dtype=jnp.int32)
    src_pad = (ar * 997) % N
    dst_pad = N + (ar % (NP - N))
    srcp = jnp.concatenate([edge_index[0], src_pad]).reshape(EP // K, K)
    dstp = jnp.concatenate([edge_index[1], dst_pad]).reshape(EP // K, K)

    h = x
    for (W, b) in ((W1, b1), (W2, b2), (W3, b3)):
        parts = _agg_sc(h, srcp, dstp)
        h = _layer_tc(h, parts, W, b)

    pmax, psum, pcnt = _pool_sc(h, batch_index)
    return _head_tc(pmax, psum, pcnt, Wl1, bl1, Wl2, bl2, Wl3, bl3)
